# Initial kernel scaffold; baseline (speedup 1.0000x reference)
import jax
import jax.numpy as jnp
from jax.experimental import pallas as pl


def kernel(x, W, b, bn_w, bn_b, edge_index):
    raise NotImplementedError("write your pallas kernel here")


# speedup vs baseline: 23.3442x; 23.3442x over previous
import functools

import jax
import jax.numpy as jnp
from jax import lax
from jax.experimental import pallas as pl
from jax.experimental.pallas import tpu as pltpu
from jax.experimental.pallas import tpu_sc as plsc

NC = 2
NS = 16
LANES = 16
NW = NC * NS
K = 128
BM = 1280


def _make_sc1(Npad, CK, NI):
    SPS = Npad // NS
    mesh = plsc.VectorSubcoreMesh(core_axis_name="c", subcore_axis_name="s")

    @functools.partial(
        pl.kernel,
        out_type=jax.ShapeDtypeStruct((NC, Npad), jnp.float32),
        mesh=mesh,
        compiler_params=pltpu.CompilerParams(needs_layout_passes=False),
        scratch_types=[
            pltpu.VMEM((CK,), jnp.int32),
            pltpu.VMEM((Npad,), jnp.float32),
            pltpu.VMEM((NI, K), jnp.int32),
            pltpu.VMEM_SHARED((Npad,), jnp.float32),
        ],
    )
    def sc1(src_hbm, zeros_hbm, ident_hbm, out_hbm, idx_v, acc_v, ident_v, sh):
        c = lax.axis_index("c")
        s = lax.axis_index("s")
        w = c * NS + s
        pltpu.sync_copy(src_hbm.at[w], idx_v)
        pltpu.sync_copy(zeros_hbm, acc_v)
        pltpu.sync_copy(ident_hbm, ident_v)
        pltpu.sync_copy(zeros_hbm.at[pl.ds(s * SPS, SPS)],
                        sh.at[pl.ds(s * SPS, SPS)])
        ones = jnp.full((LANES,), 1.0, jnp.float32)

        def body(i, carry):
            idx = idx_v[pl.ds(i * LANES, LANES)]
            plsc.addupdate_scatter(acc_v, [idx], ones)
            return carry

        lax.fori_loop(0, CK // LANES, body, 0)
        plsc.subcore_barrier()

        def comb(j, carry):
            pltpu.sync_copy(acc_v.at[pl.ds(j * K, K)], sh.at[ident_v.at[j]],
                            add=True)
            return carry

        lax.fori_loop(0, NI, comb, 0)
        plsc.subcore_barrier()
        pltpu.sync_copy(sh.at[pl.ds(s * SPS, SPS)],
                        out_hbm.at[c].at[pl.ds(s * SPS, SPS)])

    return sc1


def _make_sc2(Npad, CK, NI):
    SPS = Npad // NS
    mesh = plsc.VectorSubcoreMesh(core_axis_name="c", subcore_axis_name="s")

    @functools.partial(
        pl.kernel,
        out_type=jax.ShapeDtypeStruct((NC, Npad), jnp.float32),
        mesh=mesh,
        compiler_params=pltpu.CompilerParams(needs_layout_passes=False),
        scratch_types=[
            pltpu.VMEM((CK,), jnp.int32),
            pltpu.VMEM((CK,), jnp.int32),
            pltpu.VMEM((Npad,), jnp.float32),
            pltpu.VMEM((Npad,), jnp.float32),
            pltpu.VMEM((NI, K), jnp.int32),
            pltpu.VMEM_SHARED((Npad,), jnp.float32),
        ],
    )
    def sc2(src_hbm, dst_hbm, dis_hbm, zeros_hbm, ident_hbm, out_hbm,
            sidx_v, didx_v, dis_v, acc_v, ident_v, sh):
        c = lax.axis_index("c")
        s = lax.axis_index("s")
        w = c * NS + s
        pltpu.sync_copy(src_hbm.at[w], sidx_v)
        pltpu.sync_copy(dst_hbm.at[w], didx_v)
        pltpu.sync_copy(dis_hbm, dis_v)
        pltpu.sync_copy(zeros_hbm, acc_v)
        pltpu.sync_copy(ident_hbm, ident_v)
        pltpu.sync_copy(zeros_hbm.at[pl.ds(s * SPS, SPS)],
                        sh.at[pl.ds(s * SPS, SPS)])

        def body(i, carry):
            si = sidx_v[pl.ds(i * LANES, LANES)]
            di = didx_v[pl.ds(i * LANES, LANES)]
            vals = plsc.load_gather(dis_v, [si])
            plsc.addupdate_scatter(acc_v, [di], vals)
            return carry

        lax.fori_loop(0, CK // LANES, body, 0)
        plsc.subcore_barrier()

        def comb(j, carry):
            pltpu.sync_copy(acc_v.at[pl.ds(j * K, K)], sh.at[ident_v.at[j]],
                            add=True)
            return carry

        lax.fori_loop(0, NI, comb, 0)
        plsc.subcore_barrier()
        pltpu.sync_copy(sh.at[pl.ds(s * SPS, SPS)],
                        out_hbm.at[c].at[pl.ds(s * SPS, SPS)])

    return sc2


def _make_sc3(Npad, D, CK, C):
    ROWS = Npad // NS
    mesh = plsc.VectorSubcoreMesh(core_axis_name="c", subcore_axis_name="s")

    @functools.partial(
        pl.kernel,
        out_type=jax.ShapeDtypeStruct((NC, Npad, D), jnp.float32),
        mesh=mesh,
        compiler_params=pltpu.CompilerParams(needs_layout_passes=False),
        scratch_types=[
            pltpu.VMEM((CK,), jnp.int32),
            pltpu.VMEM((C, K), jnp.int32),
            pltpu.VMEM((K, D), jnp.float32),
            pltpu.VMEM_SHARED((Npad, D), jnp.float32),
            pltpu.SemaphoreType.DMA,
        ],
    )
    def sc3(g_hbm, src_hbm, dst3_hbm, zrows_hbm, out_hbm,
            sidx_v, didx_v, rows_v, m_sh, sem):
        c = lax.axis_index("c")
        s = lax.axis_index("s")
        w = c * NS + s
        pltpu.sync_copy(src_hbm.at[w], sidx_v)
        pltpu.sync_copy(dst3_hbm.at[w], didx_v)
        pltpu.sync_copy(zrows_hbm, rows_v)
        for kk in range(ROWS // K):
            pltpu.sync_copy(rows_v, m_sh.at[pl.ds(s * ROWS + kk * K, K)])
        plsc.subcore_barrier()

        def body(j, carry):
            pltpu.async_copy(g_hbm.at[sidx_v.at[pl.ds(j * K, K)]], rows_v,
                             sem).wait()
            pltpu.sync_copy(rows_v, m_sh.at[didx_v.at[j]], add=True)
            return carry

        lax.fori_loop(0, C, body, 0)
        plsc.subcore_barrier()
        pltpu.sync_copy(m_sh.at[pl.ds(s * ROWS, ROWS)],
                        out_hbm.at[c].at[pl.ds(s * ROWS, ROWS)])

    return sc3


def _dis_body(cnt_ref, dis_ref):
    deg = cnt_ref[0] + cnt_ref[1] + 1.0
    dis_ref[...] = lax.rsqrt(deg)


def _mm_body(x_ref, wt_ref, s0_ref, s1_ref, dis_ref,
             h_ref, g_ref, c_ref, coef_ref):
    h = jnp.dot(x_ref[...], wt_ref[...], preferred_element_type=jnp.float32)
    dis = dis_ref[...]
    deg2 = dis * (s0_ref[0] + s1_ref[0]) + dis * dis + 2.0
    dis2 = lax.rsqrt(deg2)
    cvec = dis * dis2
    h_ref[...] = h
    g_ref[...] = cvec * h
    c_ref[...] = cvec
    coef_ref[...] = cvec * cvec + 2.0 * (dis2 * dis2)


def _fin_body(m_ref, h_ref, x_ref, c_ref, coef_ref, swbb_ref, o_ref):
    agg = c_ref[...] * (m_ref[0] + m_ref[1]) + coef_ref[...] * h_ref[...]
    pre = agg * swbb_ref[0:1, :] + swbb_ref[1:2, :]
    o_ref[...] = jnp.maximum(jnp.maximum(pre, 0.0) + x_ref[...], 0.0)


def kernel(x, W, b, bn_w, bn_b, edge_index):
    N, D = x.shape
    E = edge_index.shape[1]
    Npad = -(-(N + 1) // BM) * BM
    C = -(-E // (NW * K))
    CK = C * K
    EP = NW * CK
    NI = Npad // K

    src = edge_index[0]
    dst = edge_index[1]
    pad = jnp.full((EP - E,), N, jnp.int32)
    src2 = jnp.concatenate([src, pad]).reshape(NW, CK)
    dstp = jnp.concatenate([dst, pad])
    dst2 = dstp.reshape(NW, CK)
    dst3 = dstp.reshape(NW, C, K)
    zeros_np = jnp.zeros((Npad,), jnp.float32)
    ident = jnp.arange(Npad, dtype=jnp.int32).reshape(NI, K)

    cnt = _make_sc1(Npad, CK, NI)(src2, zeros_np, ident)

    dis_col = pl.pallas_call(
        _dis_body,
        out_shape=jax.ShapeDtypeStruct((Npad, 1), jnp.float32),
    )(cnt.reshape(NC, Npad, 1))

    S = _make_sc2(Npad, CK, NI)(src2, dst2, dis_col.reshape(Npad),
                                zeros_np, ident)
    S_col = S.reshape(NC, Npad, 1)

    xp = jnp.pad(x, ((0, Npad - N), (0, 0)))
    grid = (Npad // BM,)
    col_spec = pl.BlockSpec((BM, 1), lambda i: (i, 0))
    row_spec = pl.BlockSpec((BM, D), lambda i: (i, 0))
    h, g, cvec, coef = pl.pallas_call(
        _mm_body,
        grid=grid,
        in_specs=[row_spec,
                  pl.BlockSpec((D, D), lambda i: (0, 0)),
                  pl.BlockSpec((1, BM, 1), lambda i: (0, i, 0)),
                  pl.BlockSpec((1, BM, 1), lambda i: (1, i, 0)),
                  col_spec],
        out_specs=[row_spec, row_spec, col_spec, col_spec],
        out_shape=[jax.ShapeDtypeStruct((Npad, D), jnp.float32),
                   jax.ShapeDtypeStruct((Npad, D), jnp.float32),
                   jax.ShapeDtypeStruct((Npad, 1), jnp.float32),
                   jax.ShapeDtypeStruct((Npad, 1), jnp.float32)],
    )(xp, W.T, S_col, S_col, dis_col)

    zrows = jnp.zeros((K, D), jnp.float32)
    mpart = _make_sc3(Npad, D, CK, C)(g, src2, dst3, zrows)

    sw = bn_w * (1.0 / jnp.sqrt(jnp.float32(1.0 + 1e-5)))
    swbb = jnp.stack([sw, b * sw + bn_b])
    outp = pl.pallas_call(
        _fin_body,
        grid=grid,
        in_specs=[pl.BlockSpec((NC, BM, D), lambda i: (0, i, 0)),
                  row_spec, row_spec, col_spec, col_spec,
                  pl.BlockSpec((2, D), lambda i: (0, 0))],
        out_specs=row_spec,
        out_shape=jax.ShapeDtypeStruct((Npad, D), jnp.float32),
    )(mpart, h, xp, cvec, coef, swbb)
    return outp[:N]


# speedup vs baseline: 26.2520x; 1.1246x over previous
import functools

import jax
import jax.numpy as jnp
from jax import lax
from jax.experimental import pallas as pl
from jax.experimental.pallas import tpu as pltpu
from jax.experimental.pallas import tpu_sc as plsc

NC = 2
NS = 16
LANES = 16
NW = NC * NS
K = 64
BM = 1280


def _make_sc1(Npad, CK, NI):
    SPS = Npad // NS
    mesh = plsc.VectorSubcoreMesh(core_axis_name="c", subcore_axis_name="s")

    @functools.partial(
        pl.kernel,
        out_type=jax.ShapeDtypeStruct((NC, Npad), jnp.float32),
        mesh=mesh,
        compiler_params=pltpu.CompilerParams(needs_layout_passes=False),
        scratch_types=[
            pltpu.VMEM((CK,), jnp.int32),
            pltpu.VMEM((Npad,), jnp.float32),
            pltpu.VMEM((NI, K), jnp.int32),
            pltpu.VMEM_SHARED((Npad,), jnp.float32),
        ],
    )
    def sc1(src_hbm, zeros_hbm, ident_hbm, out_hbm, idx_v, acc_v, ident_v, sh):
        c = lax.axis_index("c")
        s = lax.axis_index("s")
        w = c * NS + s
        pltpu.sync_copy(src_hbm.at[w], idx_v)
        pltpu.sync_copy(zeros_hbm, acc_v)
        pltpu.sync_copy(ident_hbm, ident_v)
        pltpu.sync_copy(zeros_hbm.at[pl.ds(s * SPS, SPS)],
                        sh.at[pl.ds(s * SPS, SPS)])
        ones = jnp.full((LANES,), 1.0, jnp.float32)

        def body(i, carry):
            idx = idx_v[pl.ds(i * LANES, LANES)]
            plsc.addupdate_scatter(acc_v, [idx], ones)
            return carry

        lax.fori_loop(0, CK // LANES, body, 0)
        plsc.subcore_barrier()

        def comb(j, carry):
            pltpu.sync_copy(acc_v.at[pl.ds(j * K, K)], sh.at[ident_v.at[j]],
                            add=True)
            return carry

        lax.fori_loop(0, NI, comb, 0)
        plsc.subcore_barrier()
        pltpu.sync_copy(sh.at[pl.ds(s * SPS, SPS)],
                        out_hbm.at[c].at[pl.ds(s * SPS, SPS)])

    return sc1


def _make_sc2(Npad, CK, NI):
    SPS = Npad // NS
    mesh = plsc.VectorSubcoreMesh(core_axis_name="c", subcore_axis_name="s")

    @functools.partial(
        pl.kernel,
        out_type=jax.ShapeDtypeStruct((NC, Npad), jnp.float32),
        mesh=mesh,
        compiler_params=pltpu.CompilerParams(needs_layout_passes=False),
        scratch_types=[
            pltpu.VMEM((CK,), jnp.int32),
            pltpu.VMEM((CK,), jnp.int32),
            pltpu.VMEM((Npad,), jnp.float32),
            pltpu.VMEM((Npad,), jnp.float32),
            pltpu.VMEM((NI, K), jnp.int32),
            pltpu.VMEM_SHARED((Npad,), jnp.float32),
        ],
    )
    def sc2(src_hbm, dst_hbm, dis_hbm, zeros_hbm, ident_hbm, out_hbm,
            sidx_v, didx_v, dis_v, acc_v, ident_v, sh):
        c = lax.axis_index("c")
        s = lax.axis_index("s")
        w = c * NS + s
        pltpu.sync_copy(src_hbm.at[w], sidx_v)
        pltpu.sync_copy(dst_hbm.at[w], didx_v)
        pltpu.sync_copy(dis_hbm, dis_v)
        pltpu.sync_copy(zeros_hbm, acc_v)
        pltpu.sync_copy(ident_hbm, ident_v)
        pltpu.sync_copy(zeros_hbm.at[pl.ds(s * SPS, SPS)],
                        sh.at[pl.ds(s * SPS, SPS)])

        def body(i, carry):
            si = sidx_v[pl.ds(i * LANES, LANES)]
            di = didx_v[pl.ds(i * LANES, LANES)]
            vals = plsc.load_gather(dis_v, [si])
            plsc.addupdate_scatter(acc_v, [di], vals)
            return carry

        lax.fori_loop(0, CK // LANES, body, 0)
        plsc.subcore_barrier()

        def comb(j, carry):
            pltpu.sync_copy(acc_v.at[pl.ds(j * K, K)], sh.at[ident_v.at[j]],
                            add=True)
            return carry

        lax.fori_loop(0, NI, comb, 0)
        plsc.subcore_barrier()
        pltpu.sync_copy(sh.at[pl.ds(s * SPS, SPS)],
                        out_hbm.at[c].at[pl.ds(s * SPS, SPS)])

    return sc2


def _make_sc3(Npad, D, CK, C):
    ROWS = Npad // NS
    mesh = plsc.VectorSubcoreMesh(core_axis_name="c", subcore_axis_name="s")

    @functools.partial(
        pl.kernel,
        out_type=jax.ShapeDtypeStruct((NC, Npad, D), jnp.float32),
        mesh=mesh,
        compiler_params=pltpu.CompilerParams(needs_layout_passes=False),
        scratch_types=[
            pltpu.VMEM((CK,), jnp.int32),
            pltpu.VMEM((C, K), jnp.int32),
            pltpu.VMEM((K, D), jnp.float32),
            pltpu.VMEM((K, D), jnp.float32),
            pltpu.SemaphoreType.DMA,
            pltpu.SemaphoreType.DMA,
            pltpu.VMEM_SHARED((Npad, D), jnp.float32),
        ],
    )
    def sc3(g_hbm, src_hbm, dst3_hbm, zrows_hbm, out_hbm,
            sidx_v, didx_v, rows_a, rows_b, sem_a, sem_b, m_sh):
        c = lax.axis_index("c")
        s = lax.axis_index("s")
        w = c * NS + s
        pltpu.sync_copy(src_hbm.at[w], sidx_v)
        pltpu.sync_copy(dst3_hbm.at[w], didx_v)
        pltpu.sync_copy(zrows_hbm, rows_a)
        for kk in range(ROWS // K):
            pltpu.sync_copy(rows_a, m_sh.at[pl.ds(s * ROWS + kk * K, K)])
        plsc.subcore_barrier()

        def gather(j, buf, sem):
            pltpu.async_copy(g_hbm.at[sidx_v.at[pl.ds(j * K, K)]], buf, sem)

        def wait_for(buf, sem):
            pltpu.make_async_copy(g_hbm.at[sidx_v.at[pl.ds(0, K)]], buf,
                                  sem).wait()

        gather(0, rows_a, sem_a)

        def body(i, carry):
            j = 2 * i
            gather(j + 1, rows_b, sem_b)
            wait_for(rows_a, sem_a)
            pltpu.sync_copy(rows_a, m_sh.at[didx_v.at[j]], add=True)
            gather(j + 2, rows_a, sem_a)
            wait_for(rows_b, sem_b)
            pltpu.sync_copy(rows_b, m_sh.at[didx_v.at[j + 1]], add=True)
            return carry

        lax.fori_loop(0, C // 2 - 1, body, 0)
        gather(C - 1, rows_b, sem_b)
        wait_for(rows_a, sem_a)
        pltpu.sync_copy(rows_a, m_sh.at[didx_v.at[C - 2]], add=True)
        wait_for(rows_b, sem_b)
        pltpu.sync_copy(rows_b, m_sh.at[didx_v.at[C - 1]], add=True)
        plsc.subcore_barrier()
        pltpu.sync_copy(m_sh.at[pl.ds(s * ROWS, ROWS)],
                        out_hbm.at[c].at[pl.ds(s * ROWS, ROWS)])

    return sc3


def _dis_body(cnt_ref, dis_ref):
    deg = cnt_ref[0] + cnt_ref[1] + 1.0
    dis_ref[...] = lax.rsqrt(deg)


def _mm_body(x_ref, wt_ref, s0_ref, s1_ref, dis_ref,
             h_ref, g_ref, c_ref, coef_ref):
    h = jnp.dot(x_ref[...], wt_ref[...], preferred_element_type=jnp.float32)
    dis = dis_ref[...]
    deg2 = dis * (s0_ref[0] + s1_ref[0]) + dis * dis + 2.0
    dis2 = lax.rsqrt(deg2)
    cvec = dis * dis2
    h_ref[...] = h
    g_ref[...] = cvec * h
    c_ref[...] = cvec
    coef_ref[...] = cvec * cvec + 2.0 * (dis2 * dis2)


def _fin_body(m_ref, h_ref, x_ref, c_ref, coef_ref, swbb_ref, o_ref):
    agg = c_ref[...] * (m_ref[0] + m_ref[1]) + coef_ref[...] * h_ref[...]
    pre = agg * swbb_ref[0:1, :] + swbb_ref[1:2, :]
    o_ref[...] = jnp.maximum(jnp.maximum(pre, 0.0) + x_ref[...], 0.0)


def kernel(x, W, b, bn_w, bn_b, edge_index):
    N, D = x.shape
    E = edge_index.shape[1]
    Npad = -(-(N + 1) // BM) * BM
    C = 2 * (-(-E // (NW * K * 2)))
    CK = C * K
    EP = NW * CK
    NI = Npad // K

    src = edge_index[0]
    dst = edge_index[1]
    pad = jnp.full((EP - E,), N, jnp.int32)
    src2 = jnp.concatenate([src, pad]).reshape(NW, CK)
    dstp = jnp.concatenate([dst, pad])
    dst2 = dstp.reshape(NW, CK)
    dst3 = dstp.reshape(NW, C, K)
    zeros_np = jnp.zeros((Npad,), jnp.float32)
    ident = jnp.arange(Npad, dtype=jnp.int32).reshape(NI, K)

    cnt = _make_sc1(Npad, CK, NI)(src2, zeros_np, ident)

    dis_col = pl.pallas_call(
        _dis_body,
        out_shape=jax.ShapeDtypeStruct((Npad, 1), jnp.float32),
    )(cnt.reshape(NC, Npad, 1))

    S = _make_sc2(Npad, CK, NI)(src2, dst2, dis_col.reshape(Npad),
                                zeros_np, ident)
    S_col = S.reshape(NC, Npad, 1)

    xp = jnp.pad(x, ((0, Npad - N), (0, 0)))
    grid = (Npad // BM,)
    col_spec = pl.BlockSpec((BM, 1), lambda i: (i, 0))
    row_spec = pl.BlockSpec((BM, D), lambda i: (i, 0))
    h, g, cvec, coef = pl.pallas_call(
        _mm_body,
        grid=grid,
        in_specs=[row_spec,
                  pl.BlockSpec((D, D), lambda i: (0, 0)),
                  pl.BlockSpec((1, BM, 1), lambda i: (0, i, 0)),
                  pl.BlockSpec((1, BM, 1), lambda i: (1, i, 0)),
                  col_spec],
        out_specs=[row_spec, row_spec, col_spec, col_spec],
        out_shape=[jax.ShapeDtypeStruct((Npad, D), jnp.float32),
                   jax.ShapeDtypeStruct((Npad, D), jnp.float32),
                   jax.ShapeDtypeStruct((Npad, 1), jnp.float32),
                   jax.ShapeDtypeStruct((Npad, 1), jnp.float32)],
    )(xp, W.T, S_col, S_col, dis_col)

    zrows = jnp.zeros((K, D), jnp.float32)
    mpart = _make_sc3(Npad, D, CK, C)(g, src2, dst3, zrows)

    sw = bn_w * (1.0 / jnp.sqrt(jnp.float32(1.0 + 1e-5)))
    swbb = jnp.stack([sw, b * sw + bn_b])
    outp = pl.pallas_call(
        _fin_body,
        grid=grid,
        in_specs=[pl.BlockSpec((NC, BM, D), lambda i: (0, i, 0)),
                  row_spec, row_spec, col_spec, col_spec,
                  pl.BlockSpec((2, D), lambda i: (0, 0))],
        out_specs=row_spec,
        out_shape=jax.ShapeDtypeStruct((Npad, D), jnp.float32),
    )(mpart, h, xp, cvec, coef, swbb)
    return outp[:N]
